# Initial kernel scaffold; baseline (speedup 1.0000x reference)
import jax
import jax.numpy as jnp
from jax.experimental import pallas as pl


def kernel(x, aq_embed, codebook):
    raise NotImplementedError("write your pallas kernel here")


# speedup vs baseline: 3.9748x; 3.9748x over previous
import functools
import math

import numpy as np
import jax
import jax.numpy as jnp
from jax import lax
from jax.experimental import pallas as pl
from jax.experimental.pallas import tpu as pltpu
from jax.experimental.pallas import tpu_sc as plsc

D_MODEL = 256
NUM_CODES = 1024
N_ROWS = 256

def _build_pe_np():
    position = np.arange(N_ROWS).reshape(-1, 1).astype(np.float32)
    div_term = np.exp(
        np.arange(0, D_MODEL, 2).astype(np.float32) * (-math.log(10000.0) / D_MODEL)
    )
    pe = np.zeros((N_ROWS, D_MODEL), dtype=np.float32)
    pe[:, 0::2] = np.sin(position * div_term)
    pe[:, 1::2] = np.cos(position * div_term)
    return pe

_PE = _build_pe_np()


def _argmin_body(x_ref, cbt_ref, idx_ref):
    x = x_ref[...]
    cbt = cbt_ref[...]
    prod = lax.dot_general(
        x, cbt, (((1,), (0,)), ((), ())),
        preferred_element_type=jnp.float32,
        precision=lax.Precision.HIGHEST,
    )
    cbn = jnp.sum(cbt * cbt, axis=0, keepdims=True)
    scores = cbn - 2.0 * prod
    m = jnp.min(scores, axis=1, keepdims=True)
    iota = lax.broadcasted_iota(jnp.int32, scores.shape, 1)
    idx = jnp.min(
        jnp.where(scores == m, iota, jnp.int32(NUM_CODES)),
        axis=1, keepdims=True,
    )
    idx_ref[...] = idx


def _compute_indices(x, cbt):
    return pl.pallas_call(
        _argmin_body,
        out_shape=jax.ShapeDtypeStruct((N_ROWS, 1), jnp.int32),
    )(x, cbt)


_info = plsc.get_sparse_core_info()
_NC = _info.num_cores
_NS = _info.num_subcores
_L = _info.num_lanes
_NW = _NC * _NS
_BPW = N_ROWS // _NW

_sc_mesh = plsc.VectorSubcoreMesh(core_axis_name="c", subcore_axis_name="s")


@functools.partial(
    pl.kernel,
    mesh=_sc_mesh,
    out_type=jax.ShapeDtypeStruct((N_ROWS, D_MODEL), jnp.float32),
    scratch_types=[
        pltpu.VMEM((_BPW,), jnp.int32),
        pltpu.VMEM((_BPW, D_MODEL), jnp.float32),
        pltpu.VMEM((_BPW, D_MODEL), jnp.float32),
        pltpu.SemaphoreType.DMA,
    ],
)
def _gather_pe(table_hbm, idx_hbm, pe_hbm, out_hbm, idx_v, rows_v, pe_v, sem):
    wid = lax.axis_index("s") * _NC + lax.axis_index("c")
    base = wid * _BPW
    pltpu.sync_copy(idx_hbm.at[pl.ds(base, _BPW)], idx_v)
    g = pltpu.async_copy(table_hbm.at[idx_v], rows_v, sem)
    pltpu.sync_copy(pe_hbm.at[pl.ds(base, _BPW)], pe_v)
    g.wait()
    for r in range(_BPW):
        for j in range(D_MODEL // _L):
            sl = pl.ds(j * _L, _L)
            rows_v[r, sl] = rows_v[r, sl] + pe_v[r, sl]
    pltpu.sync_copy(rows_v, out_hbm.at[pl.ds(base, _BPW)])


def kernel(x, aq_embed, codebook):
    cbt = codebook.T
    idx = _compute_indices(x, cbt).reshape(N_ROWS)
    pe = jnp.asarray(_PE)
    return _gather_pe(aq_embed, idx, pe)


# speedup vs baseline: 4.6497x; 1.1698x over previous
import functools
import math

import numpy as np
import jax
import jax.numpy as jnp
from jax import lax
from jax.experimental import pallas as pl
from jax.experimental.pallas import tpu as pltpu
from jax.experimental.pallas import tpu_sc as plsc

D_MODEL = 256
NUM_CODES = 1024
N_ROWS = 256

def _build_pe_np():
    position = np.arange(N_ROWS).reshape(-1, 1).astype(np.float32)
    div_term = np.exp(
        np.arange(0, D_MODEL, 2).astype(np.float32) * (-math.log(10000.0) / D_MODEL)
    )
    pe = np.zeros((N_ROWS, D_MODEL), dtype=np.float32)
    pe[:, 0::2] = np.sin(position * div_term)
    pe[:, 1::2] = np.cos(position * div_term)
    return pe

_PE = _build_pe_np()


_KBLK = 256
_NBLK = NUM_CODES // _KBLK


def _argmin_body(x_ref, cb_ref, idx_ref, minv_ref):
    k = pl.program_id(0)
    x = x_ref[...]
    cb = cb_ref[...]
    prod = lax.dot_general(
        cb, x, (((1,), (1,)), ((), ())),
        preferred_element_type=jnp.float32,
    )
    cbn = jnp.sum(cb * cb, axis=1, keepdims=True)
    scores = cbn - 2.0 * prod
    m = jnp.min(scores, axis=0, keepdims=True)
    iota = lax.broadcasted_iota(jnp.int32, scores.shape, 0) + k * _KBLK
    lidx = jnp.min(
        jnp.where(scores == m, iota, jnp.int32(NUM_CODES)),
        axis=0, keepdims=True,
    )

    @pl.when(k == 0)
    def _():
        minv_ref[...] = m
        idx_ref[...] = lidx

    @pl.when(k > 0)
    def _():
        better = m < minv_ref[...]
        minv_ref[...] = jnp.where(better, m, minv_ref[...])
        idx_ref[...] = jnp.where(better, lidx, idx_ref[...])


def _compute_indices(x, cb):
    return pl.pallas_call(
        _argmin_body,
        grid=(_NBLK,),
        in_specs=[
            pl.BlockSpec((N_ROWS, D_MODEL), lambda k: (0, 0)),
            pl.BlockSpec((_KBLK, D_MODEL), lambda k: (k, 0)),
        ],
        out_specs=pl.BlockSpec((1, N_ROWS), lambda k: (0, 0)),
        out_shape=jax.ShapeDtypeStruct((1, N_ROWS), jnp.int32),
        scratch_shapes=[pltpu.VMEM((1, N_ROWS), jnp.float32)],
    )(x, cb)


_NC = 2
_NS = 16
_L = 16
_NW = _NC * _NS
_BPW = N_ROWS // _NW

@functools.lru_cache(maxsize=None)
def _make_gather_pe():
    mesh = plsc.VectorSubcoreMesh(
        core_axis_name="c", subcore_axis_name="s", num_cores=_NC
    )
    return functools.partial(
        pl.kernel,
        mesh=mesh,
        out_type=jax.ShapeDtypeStruct((N_ROWS, D_MODEL), jnp.float32),
        scratch_types=[
            pltpu.VMEM((_BPW,), jnp.int32),
            pltpu.VMEM((_BPW, D_MODEL), jnp.float32),
            pltpu.VMEM((_BPW, D_MODEL), jnp.float32),
            pltpu.SemaphoreType.DMA,
        ],
    )(_gather_pe_body)


def _gather_pe_body(table_hbm, idx_hbm, pe_hbm, out_hbm, idx_v, rows_v, pe_v, sem):
    wid = lax.axis_index("s") * _NC + lax.axis_index("c")
    base = wid * _BPW
    pltpu.sync_copy(idx_hbm.at[0, pl.ds(base, _BPW)], idx_v)
    g = pltpu.async_copy(table_hbm.at[idx_v], rows_v, sem)
    pltpu.sync_copy(pe_hbm.at[pl.ds(base, _BPW)], pe_v)
    g.wait()
    def _add_row(r):
        def body(j, carry):
            sl = pl.ds(j * _L, _L)
            rows_v[r, sl] = rows_v[r, sl] + pe_v[r, sl]
            return carry
        lax.fori_loop(0, D_MODEL // _L, body, 0)

    for r in range(_BPW):
        _add_row(r)
    pltpu.sync_copy(rows_v, out_hbm.at[pl.ds(base, _BPW)])


def kernel(x, aq_embed, codebook):
    idx = _compute_indices(x, codebook)
    pe = jnp.asarray(_PE)
    return _make_gather_pe()(aq_embed, idx, pe)


# speedup vs baseline: 4.9119x; 1.0564x over previous
import functools
import math

import numpy as np
import jax
import jax.numpy as jnp
from jax import lax
from jax.experimental import pallas as pl
from jax.experimental.pallas import tpu as pltpu
from jax.experimental.pallas import tpu_sc as plsc

D_MODEL = 256
NUM_CODES = 1024
N_ROWS = 256

def _build_pe_np():
    position = np.arange(N_ROWS).reshape(-1, 1).astype(np.float32)
    div_term = np.exp(
        np.arange(0, D_MODEL, 2).astype(np.float32) * (-math.log(10000.0) / D_MODEL)
    )
    pe = np.zeros((N_ROWS, D_MODEL), dtype=np.float32)
    pe[:, 0::2] = np.sin(position * div_term)
    pe[:, 1::2] = np.cos(position * div_term)
    return pe

_PE = _build_pe_np()


def _argmin_body(x_ref, cb_ref, idx_ref):
    x = x_ref[...]
    cb = cb_ref[...]
    prod = lax.dot_general(
        cb, x, (((1,), (1,)), ((), ())),
        preferred_element_type=jnp.float32,
    )
    cbn = jnp.sum(cb * cb, axis=1, keepdims=True)
    scores = cbn - 2.0 * prod
    m = jnp.min(scores, axis=0, keepdims=True)
    iota = lax.broadcasted_iota(jnp.int32, scores.shape, 0)
    idx_ref[...] = jnp.min(
        jnp.where(scores == m, iota, jnp.int32(NUM_CODES)),
        axis=0, keepdims=True,
    )


def _compute_indices(x, cb):
    return pl.pallas_call(
        _argmin_body,
        out_shape=jax.ShapeDtypeStruct((1, N_ROWS), jnp.int32),
    )(x, cb)


_NC = 2
_NS = 16
_L = 16
_NW = _NC * _NS
_BPW = N_ROWS // _NW

@functools.lru_cache(maxsize=None)
def _make_gather_pe():
    mesh = plsc.VectorSubcoreMesh(
        core_axis_name="c", subcore_axis_name="s", num_cores=_NC
    )
    return functools.partial(
        pl.kernel,
        mesh=mesh,
        out_type=jax.ShapeDtypeStruct((N_ROWS, D_MODEL), jnp.float32),
        scratch_types=[
            pltpu.VMEM((_BPW,), jnp.int32),
            pltpu.VMEM((_BPW, D_MODEL), jnp.float32),
            pltpu.VMEM((_BPW, D_MODEL), jnp.float32),
            pltpu.SemaphoreType.DMA,
            pltpu.SemaphoreType.DMA,
        ],
    )(_gather_pe_body)


def _gather_pe_body(
    table_hbm, idx_hbm, pe_hbm, out_hbm, idx_v, rows_v, pe_v, sem, sem2
):
    wid = lax.axis_index("s") * _NC + lax.axis_index("c")
    base = wid * _BPW
    p = pltpu.async_copy(pe_hbm.at[pl.ds(base, _BPW)], pe_v, sem2)
    pltpu.sync_copy(idx_hbm.at[0, pl.ds(base, _BPW)], idx_v)
    g = pltpu.async_copy(table_hbm.at[idx_v], rows_v, sem)
    p.wait()
    g.wait()
    def _add_row(r):
        def body(j, carry):
            sl = pl.ds(j * _L, _L)
            rows_v[r, sl] = rows_v[r, sl] + pe_v[r, sl]
            return carry
        lax.fori_loop(0, D_MODEL // _L, body, 0)

    for r in range(_BPW):
        _add_row(r)
    pltpu.sync_copy(rows_v, out_hbm.at[pl.ds(base, _BPW)])


def kernel(x, aq_embed, codebook):
    idx = _compute_indices(x, codebook)
    pe = jnp.asarray(_PE)
    return _make_gather_pe()(aq_embed, idx, pe)
